# Initial kernel scaffold; baseline (speedup 1.0000x reference)
import jax
import jax.numpy as jnp
from jax.experimental import pallas as pl


def kernel(x, edge_index, edge_attr, is_binding_prone, W_l1, b_l1, W_r1, W_l2, b_l2, W_r2, W_l3, b_l3, W_r3, W_pre, b_pre, W_fc1, b_fc1, W_fc2, b_fc2, binding_bias):
    raise NotImplementedError("write your pallas kernel here")


# speedup vs baseline: 2.9899x; 2.9899x over previous
import jax
import jax.numpy as jnp
from jax import lax
from jax.experimental import pallas as pl
from jax.experimental.pallas import tpu as pltpu
from jax.experimental.pallas import tpu_sc as plsc

N = 10000
E = 320000
D = 128
H1 = 512
H2 = 256
H3 = 64

NPAD = 10112
EPAD = 327680
EROWS = EPAD // 128
K = 1
TROWS = NPAD // 16
WBS = (128, 128, 128, 128, 120)

_f32 = jnp.float32
_i32 = jnp.int32


def _leaky(v):
    return jnp.where(v > 0, v, 0.15 * v)


def _make_sc_agg(edge_split):
    core_rows = EROWS // 2 if edge_split else EROWS
    tile_rows = core_rows // 16

    mesh = plsc.VectorSubcoreMesh(core_axis_name="c", subcore_axis_name="s")

    out_type = [jax.ShapeDtypeStruct((2, NPAD, 128), _f32)]
    scratch = [pltpu.VMEM((1, 128), _i32),
               pltpu.VMEM((1, 128), _i32),
               pltpu.VMEM((128, 128), _f32),
               pltpu.VMEM_SHARED((NPAD, 128), _f32),
               pltpu.SemaphoreType.DMA]

    def body(tab, srcs, dsts, zer_h, out, src_v, dst_v, rows_v, acc, sem):
        cid = lax.axis_index("c")
        sid = lax.axis_index("s")
        row0 = sid * TROWS

        pltpu.sync_copy(zer_h, rows_v)
        ro = 0
        for w in WBS:
            pltpu.sync_copy(rows_v.at[pl.ds(0, w)],
                            acc.at[pl.ds(row0 + ro, w)])
            ro += w
        plsc.subcore_barrier()

        base = sid * tile_rows
        if edge_split:
            base = base + cid * core_rows

        def macro(m, carry):
            r = base + m
            pltpu.sync_copy(srcs.at[cid, pl.ds(r, 1)], src_v)
            pltpu.sync_copy(dsts.at[pl.ds(r, 1)], dst_v)
            pltpu.async_copy(tab.at[src_v.at[0]], rows_v, sem).wait()
            pltpu.sync_copy(rows_v, acc.at[dst_v.at[0]], add=True)
            return carry

        lax.fori_loop(0, tile_rows, macro, 0)
        plsc.subcore_barrier()

        ro = 0
        for w in WBS:
            pltpu.sync_copy(acc.at[pl.ds(row0 + ro, w)],
                            rows_v.at[pl.ds(0, w)])
            pltpu.sync_copy(rows_v.at[pl.ds(0, w)],
                            out.at[cid, pl.ds(row0 + ro, w)])
            ro += w

    return pl.kernel(body, out_type=out_type, mesh=mesh, scratch_types=scratch)


def _make_sc_deg():
    core_rows = EROWS // 2
    tile_rows = core_rows // 16
    mesh = plsc.VectorSubcoreMesh(core_axis_name="c", subcore_axis_name="s")
    out_type = [jax.ShapeDtypeStruct((2, NPAD, 128), _f32)]
    scratch = [pltpu.VMEM((1, 128), _i32),
               pltpu.VMEM((128, 128), _f32),
               pltpu.VMEM_SHARED((NPAD, 128), _f32)]

    def body(dsts, zer_h, ones_h, dout, dst_v, ones_v, dacc):
        cid = lax.axis_index("c")
        sid = lax.axis_index("s")
        row0 = sid * TROWS

        pltpu.sync_copy(zer_h, ones_v)
        ro = 0
        for w in WBS:
            pltpu.sync_copy(ones_v.at[pl.ds(0, w)],
                            dacc.at[pl.ds(row0 + ro, w)])
            ro += w
        pltpu.sync_copy(ones_h, ones_v)
        plsc.subcore_barrier()

        base = sid * tile_rows + cid * core_rows

        def macro(m, carry):
            pltpu.sync_copy(dsts.at[pl.ds(base + m, 1)], dst_v)
            pltpu.sync_copy(ones_v, dacc.at[dst_v.at[0]], add=True)
            return carry

        lax.fori_loop(0, tile_rows, macro, 0)
        plsc.subcore_barrier()

        ro = 0
        for w in WBS:
            pltpu.sync_copy(dacc.at[pl.ds(row0 + ro, w)],
                            ones_v.at[pl.ds(0, w)])
            pltpu.sync_copy(ones_v.at[pl.ds(0, w)],
                            dout.at[cid, pl.ds(row0 + ro, w)])
            ro += w

    return pl.kernel(body, out_type=out_type, mesh=mesh, scratch_types=scratch)


ROWB = 400
GRID = N // ROWB


def _rows(width):
    return pl.BlockSpec((ROWB, width), lambda i: (i, 0))


def _part(c, width):
    return pl.BlockSpec((1, ROWB, width), lambda i, _c=c: (_c, i, 0))


def _full(shape):
    return pl.BlockSpec(shape, lambda i: tuple(0 for _ in shape))


def _tc1_body(a0, a1, d0, d1, x, wl1, b1, wr1, wl2, wr2, y2lo, y2hi, r2):
    deg = jnp.maximum(d0[0, :, :1] + d1[0, :, :1], 1.0)
    agg = (a0[0] + a1[0]) / deg
    h1 = _leaky(jnp.dot(agg, wl1[...], preferred_element_type=_f32) + b1[...]
                + jnp.dot(x[...], wr1[...], preferred_element_type=_f32))
    y2 = jnp.dot(h1, wl2[...], preferred_element_type=_f32)
    y2lo[...] = y2[:, :128]
    y2hi[...] = y2[:, 128:]
    r2[...] = jnp.dot(h1, wr2[...], preferred_element_type=_f32)


def _tc2_body(o0, o1, d0, d1, r2, b2, wl3, wr3, y3p, r3):
    deg = jnp.maximum(d0[0, :, :1] + d1[0, :, :1], 1.0)
    agg = jnp.concatenate([o0[0], o1[0]], axis=1) / deg
    h2 = _leaky(agg + b2[...] + r2[...])
    y3 = jnp.dot(h2, wl3[...], preferred_element_type=_f32)
    y3p[...] = jnp.concatenate([y3, y3 * 0.0], axis=1)
    r3[...] = jnp.dot(h2, wr3[...], preferred_element_type=_f32)


def _tc3_body(c0, c1, d0, d1, r3, b3, wpre, bpre, wfc1, bfc1, wfc2, bfc2,
              ibp, bb, out):
    deg = jnp.maximum(d0[0, :, :1] + d1[0, :, :1], 1.0)
    agg3 = (c0[0, :, :H3] + c1[0, :, :H3]) / deg
    h3 = _leaky(agg3 + b3[...] + r3[...])
    h4 = jnp.dot(h3, wpre[...], preferred_element_type=_f32) + bpre[...]
    z = _leaky(jnp.dot(h4, wfc1[...], preferred_element_type=_f32) + bfc1[...])
    lg = jnp.dot(z, wfc2[...], preferred_element_type=_f32) + bfc2[...]
    col = bb[0, 0] * ibp[:, :1]
    colsel = lax.broadcasted_iota(_i32, (1, 2), 1) == 1
    out[...] = lg + jnp.where(colsel, col, 0.0)


def kernel(x, edge_index, edge_attr, is_binding_prone,
           W_l1, b_l1, W_r1, W_l2, b_l2, W_r2, W_l3, b_l3, W_r3,
           W_pre, b_pre, W_fc1, b_fc1, W_fc2, b_fc2, binding_bias):
    src = edge_index[0]
    dst = edge_index[1]
    pad = EPAD - E
    src2d = jnp.concatenate([src, jnp.zeros((pad,), _i32)]).reshape(EROWS, 128)
    dst2d = jnp.concatenate(
        [dst, jnp.full((pad,), NPAD - 1, _i32)]).reshape(EROWS, 128)
    srcs_same = jnp.stack([src2d, src2d])
    srcs_off = jnp.stack([src2d, src2d + N])

    zeros128 = jnp.zeros((128, 128), _f32)
    ones128 = jnp.ones((128, 128), _f32)

    sc_edge = _make_sc_agg(edge_split=True)
    sc_feat = _make_sc_agg(edge_split=False)

    (deg,) = _make_sc_deg()(dst2d, zeros128, ones128)
    (a,) = sc_edge(x, srcs_same, dst2d, zeros128)

    tc1 = pl.pallas_call(
        _tc1_body,
        grid=(GRID,),
        in_specs=[_part(0, 128), _part(1, 128), _part(0, 128), _part(1, 128),
                  _rows(128),
                  _full((D, H1)), _full((1, H1)), _full((D, H1)),
                  _full((H1, H2)), _full((H1, H2))],
        out_specs=[_rows(128), _rows(128), _rows(H2)],
        out_shape=[jax.ShapeDtypeStruct((N, 128), _f32),
                   jax.ShapeDtypeStruct((N, 128), _f32),
                   jax.ShapeDtypeStruct((N, H2), _f32)],
    )
    y2lo, y2hi, r2 = tc1(a, a, deg, deg, x, W_l1, b_l1.reshape(1, H1),
                         W_r1, W_l2, W_r2)
    y2s = jnp.concatenate([y2lo, y2hi], axis=0)

    (o,) = sc_feat(y2s, srcs_off, dst2d, zeros128)

    tc2 = pl.pallas_call(
        _tc2_body,
        grid=(GRID,),
        in_specs=[_part(0, 128), _part(1, 128), _part(0, 128), _part(1, 128),
                  _rows(H2), _full((1, H2)), _full((H2, H3)),
                  _full((H2, H3))],
        out_specs=[_rows(128), _rows(H3)],
        out_shape=[jax.ShapeDtypeStruct((N, 128), _f32),
                   jax.ShapeDtypeStruct((N, H3), _f32)],
    )
    y3p, r3 = tc2(o, o, deg, deg, r2, b_l2.reshape(1, H2), W_l3, W_r3)

    (c,) = sc_edge(y3p, srcs_same, dst2d, zeros128)

    tc3 = pl.pallas_call(
        _tc3_body,
        grid=(GRID,),
        in_specs=[_part(0, 128), _part(1, 128), _part(0, 128), _part(1, 128),
                  _rows(H3),
                  _full((1, H3)), _full((H3, 32)), _full((1, 32)),
                  _full((32, 32)), _full((1, 32)), _full((32, 2)),
                  _full((1, 2)), _rows(1), _full((1, 1))],
        out_specs=_rows(2),
        out_shape=jax.ShapeDtypeStruct((N, 2), _f32),
    )
    logits = tc3(c, c, deg, deg, r3, b_l3.reshape(1, H3),
                 W_pre, b_pre.reshape(1, 32), W_fc1, b_fc1.reshape(1, 32),
                 W_fc2, b_fc2.reshape(1, 2),
                 is_binding_prone.reshape(N, 1), binding_bias.reshape(1, 1))
    return logits


# speedup vs baseline: 3.5504x; 1.1875x over previous
import jax
import jax.numpy as jnp
from jax import lax
from jax.experimental import pallas as pl
from jax.experimental.pallas import tpu as pltpu
from jax.experimental.pallas import tpu_sc as plsc

N = 10000
E = 320000
D = 128
H1 = 512
H2 = 256
H3 = 64

NPAD = 10112
EPAD = 327680
EROWS = EPAD // 128
K = 1
TROWS = NPAD // 16
WBS = (128, 128, 128, 128, 120)

_f32 = jnp.float32
_i32 = jnp.int32


def _leaky(v):
    return jnp.where(v > 0, v, 0.15 * v)


def _make_sc_agg(edge_split):
    core_rows = EROWS // 2 if edge_split else EROWS
    tile_rows = core_rows // 16

    mesh = plsc.VectorSubcoreMesh(core_axis_name="c", subcore_axis_name="s")

    out_type = [jax.ShapeDtypeStruct((2, NPAD, 128), _f32)]
    scratch = [pltpu.VMEM((2, 128), _i32),
               pltpu.VMEM((2, 128), _i32),
               pltpu.VMEM((256, 128), _f32),
               pltpu.VMEM_SHARED((NPAD, 128), _f32),
               pltpu.SemaphoreType.DMA]

    def body(tab, srcs, dsts, zer_h, out, src_v, dst_v, rows_v, acc, sem):
        cid = lax.axis_index("c")
        sid = lax.axis_index("s")
        row0 = sid * TROWS

        def half(b):
            return rows_v.at[pl.ds(b * 128, 128)]

        pltpu.sync_copy(zer_h, half(0))
        ro = 0
        for w in WBS:
            pltpu.sync_copy(rows_v.at[pl.ds(0, w)],
                            acc.at[pl.ds(row0 + ro, w)])
            ro += w
        plsc.subcore_barrier()

        base = sid * tile_rows
        if edge_split:
            base = base + cid * core_rows
        last_r = base + tile_rows - 1

        pltpu.sync_copy(srcs.at[cid, pl.ds(base, 1)], src_v.at[pl.ds(0, 1)])
        pltpu.sync_copy(dsts.at[pl.ds(base, 1)], dst_v.at[pl.ds(0, 1)])
        pltpu.async_copy(tab.at[src_v.at[0]], half(0), sem)

        def pair(g, carry):
            for b in (0, 1):
                m = 2 * g + b
                nb = 1 - b
                rn = jnp.minimum(base + m + 1, last_r)
                pltpu.sync_copy(srcs.at[cid, pl.ds(rn, 1)],
                                src_v.at[pl.ds(nb, 1)])
                pltpu.sync_copy(dsts.at[pl.ds(rn, 1)],
                                dst_v.at[pl.ds(nb, 1)])
                pltpu.make_async_copy(tab.at[src_v.at[b]], half(b),
                                      sem).wait()
                pltpu.async_copy(tab.at[src_v.at[nb]], half(nb), sem)
                pltpu.sync_copy(half(b), acc.at[dst_v.at[b]], add=True)
            return carry

        lax.fori_loop(0, tile_rows // 2, pair, 0)
        pltpu.make_async_copy(tab.at[src_v.at[0]], half(0), sem).wait()
        plsc.subcore_barrier()

        ro = 0
        for w in WBS:
            pltpu.sync_copy(acc.at[pl.ds(row0 + ro, w)],
                            rows_v.at[pl.ds(0, w)])
            pltpu.sync_copy(rows_v.at[pl.ds(0, w)],
                            out.at[cid, pl.ds(row0 + ro, w)])
            ro += w

    return pl.kernel(body, out_type=out_type, mesh=mesh, scratch_types=scratch)


def _make_sc_deg():
    core_rows = EROWS // 2
    tile_rows = core_rows // 16
    mesh = plsc.VectorSubcoreMesh(core_axis_name="c", subcore_axis_name="s")
    out_type = [jax.ShapeDtypeStruct((2, NPAD, 128), _f32)]
    scratch = [pltpu.VMEM((1, 128), _i32),
               pltpu.VMEM((128, 128), _f32),
               pltpu.VMEM_SHARED((NPAD, 128), _f32)]

    def body(dsts, zer_h, ones_h, dout, dst_v, ones_v, dacc):
        cid = lax.axis_index("c")
        sid = lax.axis_index("s")
        row0 = sid * TROWS

        pltpu.sync_copy(zer_h, ones_v)
        ro = 0
        for w in WBS:
            pltpu.sync_copy(ones_v.at[pl.ds(0, w)],
                            dacc.at[pl.ds(row0 + ro, w)])
            ro += w
        pltpu.sync_copy(ones_h, ones_v)
        plsc.subcore_barrier()

        base = sid * tile_rows + cid * core_rows

        def macro(m, carry):
            pltpu.sync_copy(dsts.at[pl.ds(base + m, 1)], dst_v)
            pltpu.sync_copy(ones_v, dacc.at[dst_v.at[0]], add=True)
            return carry

        lax.fori_loop(0, tile_rows, macro, 0)
        plsc.subcore_barrier()

        ro = 0
        for w in WBS:
            pltpu.sync_copy(dacc.at[pl.ds(row0 + ro, w)],
                            ones_v.at[pl.ds(0, w)])
            pltpu.sync_copy(ones_v.at[pl.ds(0, w)],
                            dout.at[cid, pl.ds(row0 + ro, w)])
            ro += w

    return pl.kernel(body, out_type=out_type, mesh=mesh, scratch_types=scratch)


ROWB = 400
GRID = N // ROWB


def _rows(width):
    return pl.BlockSpec((ROWB, width), lambda i: (i, 0))


def _part(c, width):
    return pl.BlockSpec((1, ROWB, width), lambda i, _c=c: (_c, i, 0))


def _full(shape):
    return pl.BlockSpec(shape, lambda i: tuple(0 for _ in shape))


def _tc1_body(a0, a1, d0, d1, x, wl1, b1, wr1, wl2, wr2, y2lo, y2hi, r2):
    deg = jnp.maximum(d0[0, :, :1] + d1[0, :, :1], 1.0)
    agg = (a0[0] + a1[0]) / deg
    h1 = _leaky(jnp.dot(agg, wl1[...], preferred_element_type=_f32) + b1[...]
                + jnp.dot(x[...], wr1[...], preferred_element_type=_f32))
    y2 = jnp.dot(h1, wl2[...], preferred_element_type=_f32)
    y2lo[...] = y2[:, :128]
    y2hi[...] = y2[:, 128:]
    r2[...] = jnp.dot(h1, wr2[...], preferred_element_type=_f32)


def _tc2_body(o0, o1, d0, d1, r2, b2, wl3, wr3, y3p, r3):
    deg = jnp.maximum(d0[0, :, :1] + d1[0, :, :1], 1.0)
    agg = jnp.concatenate([o0[0], o1[0]], axis=1) / deg
    h2 = _leaky(agg + b2[...] + r2[...])
    y3 = jnp.dot(h2, wl3[...], preferred_element_type=_f32)
    y3p[...] = jnp.concatenate([y3, y3 * 0.0], axis=1)
    r3[...] = jnp.dot(h2, wr3[...], preferred_element_type=_f32)


def _tc3_body(c0, c1, d0, d1, r3, b3, wpre, bpre, wfc1, bfc1, wfc2, bfc2,
              ibp, bb, out):
    deg = jnp.maximum(d0[0, :, :1] + d1[0, :, :1], 1.0)
    agg3 = (c0[0, :, :H3] + c1[0, :, :H3]) / deg
    h3 = _leaky(agg3 + b3[...] + r3[...])
    h4 = jnp.dot(h3, wpre[...], preferred_element_type=_f32) + bpre[...]
    z = _leaky(jnp.dot(h4, wfc1[...], preferred_element_type=_f32) + bfc1[...])
    lg = jnp.dot(z, wfc2[...], preferred_element_type=_f32) + bfc2[...]
    col = bb[0, 0] * ibp[:, :1]
    colsel = lax.broadcasted_iota(_i32, (1, 2), 1) == 1
    out[...] = lg + jnp.where(colsel, col, 0.0)


def kernel(x, edge_index, edge_attr, is_binding_prone,
           W_l1, b_l1, W_r1, W_l2, b_l2, W_r2, W_l3, b_l3, W_r3,
           W_pre, b_pre, W_fc1, b_fc1, W_fc2, b_fc2, binding_bias):
    src = edge_index[0]
    dst = edge_index[1]
    pad = EPAD - E
    src2d = jnp.concatenate([src, jnp.zeros((pad,), _i32)]).reshape(EROWS, 128)
    dst2d = jnp.concatenate(
        [dst, jnp.full((pad,), NPAD - 1, _i32)]).reshape(EROWS, 128)
    srcs_same = jnp.stack([src2d, src2d])
    srcs_off = jnp.stack([src2d, src2d + N])

    zeros128 = jnp.zeros((128, 128), _f32)
    ones128 = jnp.ones((128, 128), _f32)

    sc_edge = _make_sc_agg(edge_split=True)
    sc_feat = _make_sc_agg(edge_split=False)

    (deg,) = _make_sc_deg()(dst2d, zeros128, ones128)
    (a,) = sc_edge(x, srcs_same, dst2d, zeros128)

    tc1 = pl.pallas_call(
        _tc1_body,
        grid=(GRID,),
        in_specs=[_part(0, 128), _part(1, 128), _part(0, 128), _part(1, 128),
                  _rows(128),
                  _full((D, H1)), _full((1, H1)), _full((D, H1)),
                  _full((H1, H2)), _full((H1, H2))],
        out_specs=[_rows(128), _rows(128), _rows(H2)],
        out_shape=[jax.ShapeDtypeStruct((N, 128), _f32),
                   jax.ShapeDtypeStruct((N, 128), _f32),
                   jax.ShapeDtypeStruct((N, H2), _f32)],
    )
    y2lo, y2hi, r2 = tc1(a, a, deg, deg, x, W_l1, b_l1.reshape(1, H1),
                         W_r1, W_l2, W_r2)
    y2s = jnp.concatenate([y2lo, y2hi], axis=0)

    (o,) = sc_feat(y2s, srcs_off, dst2d, zeros128)

    tc2 = pl.pallas_call(
        _tc2_body,
        grid=(GRID,),
        in_specs=[_part(0, 128), _part(1, 128), _part(0, 128), _part(1, 128),
                  _rows(H2), _full((1, H2)), _full((H2, H3)),
                  _full((H2, H3))],
        out_specs=[_rows(128), _rows(H3)],
        out_shape=[jax.ShapeDtypeStruct((N, 128), _f32),
                   jax.ShapeDtypeStruct((N, H3), _f32)],
    )
    y3p, r3 = tc2(o, o, deg, deg, r2, b_l2.reshape(1, H2), W_l3, W_r3)

    (c,) = sc_edge(y3p, srcs_same, dst2d, zeros128)

    tc3 = pl.pallas_call(
        _tc3_body,
        grid=(GRID,),
        in_specs=[_part(0, 128), _part(1, 128), _part(0, 128), _part(1, 128),
                  _rows(H3),
                  _full((1, H3)), _full((H3, 32)), _full((1, 32)),
                  _full((32, 32)), _full((1, 32)), _full((32, 2)),
                  _full((1, 2)), _rows(1), _full((1, 1))],
        out_specs=_rows(2),
        out_shape=jax.ShapeDtypeStruct((N, 2), _f32),
    )
    logits = tc3(c, c, deg, deg, r3, b_l3.reshape(1, H3),
                 W_pre, b_pre.reshape(1, 32), W_fc1, b_fc1.reshape(1, 32),
                 W_fc2, b_fc2.reshape(1, 2),
                 is_binding_prone.reshape(N, 1), binding_bias.reshape(1, 1))
    return logits


# speedup vs baseline: 3.5899x; 1.0111x over previous
import jax
import jax.numpy as jnp
from jax import lax
from jax.experimental import pallas as pl
from jax.experimental.pallas import tpu as pltpu
from jax.experimental.pallas import tpu_sc as plsc

N = 10000
E = 320000
D = 128
H1 = 512
H2 = 256
H3 = 64

NPAD = 10112
EPAD = 327680
EROWS = EPAD // 128
K = 1
TROWS = NPAD // 16
WBS = (128, 128, 128, 128, 120)

_f32 = jnp.float32
_i32 = jnp.int32


def _leaky(v):
    return jnp.where(v > 0, v, 0.15 * v)


def _make_sc_agg(edge_split):
    core_rows = EROWS // 2 if edge_split else EROWS
    tile_rows = core_rows // 16

    mesh = plsc.VectorSubcoreMesh(core_axis_name="c", subcore_axis_name="s")

    out_type = [jax.ShapeDtypeStruct((2, NPAD, 128), _f32)]
    scratch = [pltpu.VMEM((2, 128), _i32),
               pltpu.VMEM((2, 128), _i32),
               pltpu.VMEM((256, 128), _f32),
               pltpu.VMEM_SHARED((NPAD, 128), _f32),
               pltpu.SemaphoreType.DMA,
               pltpu.SemaphoreType.DMA]

    def body(tab, srcs, dsts, zer_h, out, src_v, dst_v, rows_v, acc,
             gsem, ssem):
        cid = lax.axis_index("c")
        sid = lax.axis_index("s")
        row0 = sid * TROWS

        def half(b):
            return rows_v.at[pl.ds(b * 128, 128)]

        def load_idx(slot, r):
            pltpu.sync_copy(srcs.at[cid, pl.ds(r, 1)],
                            src_v.at[pl.ds(slot, 1)])
            pltpu.sync_copy(dsts.at[pl.ds(r, 1)],
                            dst_v.at[pl.ds(slot, 1)])

        def gather(b):
            pltpu.async_copy(tab.at[src_v.at[b]], half(b), gsem)

        def gather_wait(b):
            pltpu.make_async_copy(tab.at[src_v.at[b]], half(b), gsem).wait()

        def scatter(b):
            pltpu.async_copy(half(b), acc.at[dst_v.at[b]], ssem, add=True)

        def scatter_wait(b):
            pltpu.make_async_copy(half(b), acc.at[dst_v.at[b]], ssem).wait()

        pltpu.sync_copy(zer_h, half(0))
        ro = 0
        for w in WBS:
            pltpu.sync_copy(rows_v.at[pl.ds(0, w)],
                            acc.at[pl.ds(row0 + ro, w)])
            ro += w
        plsc.subcore_barrier()

        base = sid * tile_rows
        if edge_split:
            base = base + cid * core_rows
        last_r = base + tile_rows - 1

        load_idx(0, base)
        gather(0)
        load_idx(1, base + 1)
        gather_wait(0)
        gather(1)
        scatter(0)

        def pair(g, carry):
            for b, moff in ((1, 1), (0, 2)):
                m = 2 * g + moff
                nb = 1 - b
                scatter_wait(nb)
                rn = jnp.minimum(base + m + 1, last_r)
                load_idx(nb, rn)
                gather_wait(b)
                gather(nb)
                scatter(b)
            return carry

        lax.fori_loop(0, tile_rows // 2 - 1, pair, 0)
        scatter_wait(0)
        gather_wait(1)
        scatter(1)
        scatter_wait(1)
        plsc.subcore_barrier()

        ro = 0
        for w in WBS:
            pltpu.sync_copy(acc.at[pl.ds(row0 + ro, w)],
                            rows_v.at[pl.ds(0, w)])
            pltpu.sync_copy(rows_v.at[pl.ds(0, w)],
                            out.at[cid, pl.ds(row0 + ro, w)])
            ro += w

    return pl.kernel(body, out_type=out_type, mesh=mesh, scratch_types=scratch)


def _make_sc_deg():
    core_rows = EROWS // 2
    tile_rows = core_rows // 16
    mesh = plsc.VectorSubcoreMesh(core_axis_name="c", subcore_axis_name="s")
    out_type = [jax.ShapeDtypeStruct((2, NPAD, 128), _f32)]
    scratch = [pltpu.VMEM((1, 128), _i32),
               pltpu.VMEM((128, 128), _f32),
               pltpu.VMEM_SHARED((NPAD, 128), _f32)]

    def body(dsts, zer_h, ones_h, dout, dst_v, ones_v, dacc):
        cid = lax.axis_index("c")
        sid = lax.axis_index("s")
        row0 = sid * TROWS

        pltpu.sync_copy(zer_h, ones_v)
        ro = 0
        for w in WBS:
            pltpu.sync_copy(ones_v.at[pl.ds(0, w)],
                            dacc.at[pl.ds(row0 + ro, w)])
            ro += w
        pltpu.sync_copy(ones_h, ones_v)
        plsc.subcore_barrier()

        base = sid * tile_rows + cid * core_rows

        def macro(m, carry):
            pltpu.sync_copy(dsts.at[pl.ds(base + m, 1)], dst_v)
            pltpu.sync_copy(ones_v, dacc.at[dst_v.at[0]], add=True)
            return carry

        lax.fori_loop(0, tile_rows, macro, 0)
        plsc.subcore_barrier()

        ro = 0
        for w in WBS:
            pltpu.sync_copy(dacc.at[pl.ds(row0 + ro, w)],
                            ones_v.at[pl.ds(0, w)])
            pltpu.sync_copy(ones_v.at[pl.ds(0, w)],
                            dout.at[cid, pl.ds(row0 + ro, w)])
            ro += w

    return pl.kernel(body, out_type=out_type, mesh=mesh, scratch_types=scratch)


ROWB = 400
GRID = N // ROWB


def _rows(width):
    return pl.BlockSpec((ROWB, width), lambda i: (i, 0))


def _part(c, width):
    return pl.BlockSpec((1, ROWB, width), lambda i, _c=c: (_c, i, 0))


def _full(shape):
    return pl.BlockSpec(shape, lambda i: tuple(0 for _ in shape))


def _tc1_body(a0, a1, d0, d1, x, wl1, b1, wr1, wl2, wr2, y2lo, y2hi, r2):
    deg = jnp.maximum(d0[0, :, :1] + d1[0, :, :1], 1.0)
    agg = (a0[0] + a1[0]) / deg
    h1 = _leaky(jnp.dot(agg, wl1[...], preferred_element_type=_f32) + b1[...]
                + jnp.dot(x[...], wr1[...], preferred_element_type=_f32))
    y2 = jnp.dot(h1, wl2[...], preferred_element_type=_f32)
    y2lo[...] = y2[:, :128]
    y2hi[...] = y2[:, 128:]
    r2[...] = jnp.dot(h1, wr2[...], preferred_element_type=_f32)


def _tc2_body(o0, o1, d0, d1, r2, b2, wl3, wr3, y3p, r3):
    deg = jnp.maximum(d0[0, :, :1] + d1[0, :, :1], 1.0)
    agg = jnp.concatenate([o0[0], o1[0]], axis=1) / deg
    h2 = _leaky(agg + b2[...] + r2[...])
    y3 = jnp.dot(h2, wl3[...], preferred_element_type=_f32)
    y3p[...] = jnp.concatenate([y3, y3 * 0.0], axis=1)
    r3[...] = jnp.dot(h2, wr3[...], preferred_element_type=_f32)


def _tc3_body(c0, c1, d0, d1, r3, b3, wpre, bpre, wfc1, bfc1, wfc2, bfc2,
              ibp, bb, out):
    deg = jnp.maximum(d0[0, :, :1] + d1[0, :, :1], 1.0)
    agg3 = (c0[0, :, :H3] + c1[0, :, :H3]) / deg
    h3 = _leaky(agg3 + b3[...] + r3[...])
    h4 = jnp.dot(h3, wpre[...], preferred_element_type=_f32) + bpre[...]
    z = _leaky(jnp.dot(h4, wfc1[...], preferred_element_type=_f32) + bfc1[...])
    lg = jnp.dot(z, wfc2[...], preferred_element_type=_f32) + bfc2[...]
    col = bb[0, 0] * ibp[:, :1]
    colsel = lax.broadcasted_iota(_i32, (1, 2), 1) == 1
    out[...] = lg + jnp.where(colsel, col, 0.0)


def kernel(x, edge_index, edge_attr, is_binding_prone,
           W_l1, b_l1, W_r1, W_l2, b_l2, W_r2, W_l3, b_l3, W_r3,
           W_pre, b_pre, W_fc1, b_fc1, W_fc2, b_fc2, binding_bias):
    src = edge_index[0]
    dst = edge_index[1]
    pad = EPAD - E
    src2d = jnp.concatenate([src, jnp.zeros((pad,), _i32)]).reshape(EROWS, 128)
    dst2d = jnp.concatenate(
        [dst, jnp.full((pad,), NPAD - 1, _i32)]).reshape(EROWS, 128)
    srcs_same = jnp.stack([src2d, src2d])
    srcs_off = jnp.stack([src2d, src2d + N])

    zeros128 = jnp.zeros((128, 128), _f32)
    ones128 = jnp.ones((128, 128), _f32)

    sc_edge = _make_sc_agg(edge_split=True)
    sc_feat = _make_sc_agg(edge_split=False)

    (deg,) = _make_sc_deg()(dst2d, zeros128, ones128)
    (a,) = sc_edge(x, srcs_same, dst2d, zeros128)

    tc1 = pl.pallas_call(
        _tc1_body,
        grid=(GRID,),
        in_specs=[_part(0, 128), _part(1, 128), _part(0, 128), _part(1, 128),
                  _rows(128),
                  _full((D, H1)), _full((1, H1)), _full((D, H1)),
                  _full((H1, H2)), _full((H1, H2))],
        out_specs=[_rows(128), _rows(128), _rows(H2)],
        out_shape=[jax.ShapeDtypeStruct((N, 128), _f32),
                   jax.ShapeDtypeStruct((N, 128), _f32),
                   jax.ShapeDtypeStruct((N, H2), _f32)],
    )
    y2lo, y2hi, r2 = tc1(a, a, deg, deg, x, W_l1, b_l1.reshape(1, H1),
                         W_r1, W_l2, W_r2)
    y2s = jnp.concatenate([y2lo, y2hi], axis=0)

    (o,) = sc_feat(y2s, srcs_off, dst2d, zeros128)

    tc2 = pl.pallas_call(
        _tc2_body,
        grid=(GRID,),
        in_specs=[_part(0, 128), _part(1, 128), _part(0, 128), _part(1, 128),
                  _rows(H2), _full((1, H2)), _full((H2, H3)),
                  _full((H2, H3))],
        out_specs=[_rows(128), _rows(H3)],
        out_shape=[jax.ShapeDtypeStruct((N, 128), _f32),
                   jax.ShapeDtypeStruct((N, H3), _f32)],
    )
    y3p, r3 = tc2(o, o, deg, deg, r2, b_l2.reshape(1, H2), W_l3, W_r3)

    (c,) = sc_edge(y3p, srcs_same, dst2d, zeros128)

    tc3 = pl.pallas_call(
        _tc3_body,
        grid=(GRID,),
        in_specs=[_part(0, 128), _part(1, 128), _part(0, 128), _part(1, 128),
                  _rows(H3),
                  _full((1, H3)), _full((H3, 32)), _full((1, 32)),
                  _full((32, 32)), _full((1, 32)), _full((32, 2)),
                  _full((1, 2)), _rows(1), _full((1, 1))],
        out_specs=_rows(2),
        out_shape=jax.ShapeDtypeStruct((N, 2), _f32),
    )
    logits = tc3(c, c, deg, deg, r3, b_l3.reshape(1, H3),
                 W_pre, b_pre.reshape(1, 32), W_fc1, b_fc1.reshape(1, 32),
                 W_fc2, b_fc2.reshape(1, 2),
                 is_binding_prone.reshape(N, 1), binding_bias.reshape(1, 1))
    return logits
